# baseline (speedup 1.0000x reference)
import functools

import jax
import jax.numpy as jnp
from jax import lax
from jax.experimental import pallas as pl
from jax.experimental.pallas import tpu as pltpu
from jax.experimental.pallas import tpu_sc as plsc

VOCAB = 100000
EMB_DIM = 128
NUM_CLASSES = 64


_ROWS_PER_BLOCK = 4000


def _project_body(emb_ref, w_ref, b_ref, out_ref):
    out_ref[...] = (
        jnp.dot(emb_ref[...], w_ref[...], preferred_element_type=jnp.float32)
        + b_ref[...]
    )


def _project(emb_table, W, b2):
    n_blocks = VOCAB // _ROWS_PER_BLOCK
    return pl.pallas_call(
        _project_body,
        grid=(n_blocks,),
        in_specs=[
            pl.BlockSpec((_ROWS_PER_BLOCK, EMB_DIM), lambda i: (i, 0)),
            pl.BlockSpec((EMB_DIM, 2 * NUM_CLASSES), lambda i: (0, 0)),
            pl.BlockSpec((1, 2 * NUM_CLASSES), lambda i: (0, 0)),
        ],
        out_specs=pl.BlockSpec(
            (_ROWS_PER_BLOCK, 2 * NUM_CLASSES), lambda i: (i, 0)
        ),
        out_shape=jax.ShapeDtypeStruct((VOCAB, 2 * NUM_CLASSES), jnp.float32),
    )(emb_table, W, b2)


_BG = 8
_LC = 20
_TOK = _BG * _LC


def _make_gather(B, L, nw):
    n_ids = B * L
    ids_per_w = n_ids // nw
    n_chunks = ids_per_w // _TOK
    assert n_chunks % 2 == 0
    lpb = L // _LC
    row_w = L * NUM_CLASSES
    seg = _LC * NUM_CLASSES
    mesh = plsc.VectorSubcoreMesh(core_axis_name="c", subcore_axis_name="s")
    nc = mesh.num_cores

    @functools.partial(
        pl.kernel,
        mesh=mesh,
        out_type=jax.ShapeDtypeStruct((B, row_w), jnp.float32),
        scratch_types=[
            pltpu.VMEM((ids_per_w,), jnp.int32),
            pltpu.VMEM((_TOK, 2 * NUM_CLASSES), jnp.float32),
            pltpu.VMEM((_TOK, 2 * NUM_CLASSES), jnp.float32),
            pltpu.VMEM((_BG, seg), jnp.float32),
            pltpu.VMEM((_BG, seg), jnp.float32),
            pltpu.SemaphoreType.DMA,
            pltpu.SemaphoreType.DMA,
            pltpu.SemaphoreType.DMA,
            pltpu.SemaphoreType.DMA,
        ],
    )
    def gather_k(
        ids_hbm, p_hbm, out_hbm,
        idx_v, gbuf0, gbuf1, obuf0, obuf1, gsem0, gsem1, ssem0, ssem1,
    ):
        wid = lax.axis_index("s") * nc + lax.axis_index("c")
        pltpu.sync_copy(ids_hbm.at[wid], idx_v)

        def start_gather(j, gbuf, gsem):
            pltpu.async_copy(
                p_hbm.at[idx_v.at[pl.ds(_TOK * j, 128)]],
                gbuf.at[pl.ds(0, 128)], gsem,
            )
            pltpu.async_copy(
                p_hbm.at[idx_v.at[pl.ds(_TOK * j + 128, _TOK - 128)]],
                gbuf.at[pl.ds(128, _TOK - 128)], gsem,
            )

        def wait_gather(j, gbuf, gsem):
            pltpu.make_async_copy(
                p_hbm.at[idx_v.at[pl.ds(_TOK * j, 128)]],
                gbuf.at[pl.ds(0, 128)], gsem,
            ).wait()
            pltpu.make_async_copy(
                p_hbm.at[idx_v.at[pl.ds(_TOK * j + 128, _TOK - 128)]],
                gbuf.at[pl.ds(128, _TOK - 128)], gsem,
            ).wait()

        def out_slice(j):
            g = wid * n_chunks + j
            b0 = (g // lpb) * _BG
            return out_hbm.at[
                pl.ds(b0, _BG), pl.ds((g % lpb) * seg, seg)
            ]

        def linearize(gbuf_t, obuf_t):
            for bi in range(_BG):
                def rowc(dl, c, bi=bi):
                    for k in range(NUM_CLASSES // 16):
                        obuf_t[bi, pl.ds(dl * NUM_CLASSES + 16 * k, 16)] = (
                            gbuf_t[bi * _LC + dl, pl.ds(16 * k, 16)]
                        )
                    return c

                lax.fori_loop(0, _LC, rowc, 0, unroll=5)

        start_gather(0, gbuf0, gsem0)
        start_gather(1, gbuf1, gsem1)

        def half_step(i, j, gbuf, obuf, gsem, ssem):
            wait_gather(j, gbuf, gsem)

            @pl.when(i > 0)
            def _():
                pltpu.make_async_copy(obuf, out_slice(j - 2), ssem).wait()

            linearize(gbuf, obuf)

            @pl.when(j + 2 < n_chunks)
            def _():
                start_gather(j + 2, gbuf, gsem)

            pltpu.async_copy(obuf, out_slice(j), ssem)

        def body(i, carry):
            j = 2 * i
            half_step(i, j, gbuf0, obuf0, gsem0, ssem0)
            half_step(i, j + 1, gbuf1, obuf1, gsem1, ssem1)
            return carry

        lax.fori_loop(0, n_chunks // 2, body, 0)
        pltpu.make_async_copy(obuf0, out_slice(n_chunks - 2), ssem0).wait()
        pltpu.make_async_copy(obuf1, out_slice(n_chunks - 1), ssem1).wait()

    return gather_k


def kernel(token_ids, emb_table, W, b):
    B, L = token_ids.shape
    info = plsc.get_sparse_core_info()
    nw = info.num_cores * info.num_subcores

    W2 = jnp.concatenate([W, W], axis=1)
    b2 = jnp.concatenate([b, b]).reshape(1, 2 * NUM_CLASSES)
    proj = _project(emb_table, W2, b2)

    ids2 = (
        token_ids.reshape(B // _BG, _BG, L // _LC, _LC)
        .swapaxes(1, 2)
        .reshape(nw, (B // nw) * L)
        .astype(jnp.int32)
    )
    out2 = _make_gather(B, L, nw)(ids2, proj)
    return out2.T.reshape(L, NUM_CLASSES, B).transpose(2, 0, 1)


# speedup vs baseline: 1.1549x; 1.1549x over previous
import functools

import jax
import jax.numpy as jnp
from jax import lax
from jax.experimental import pallas as pl
from jax.experimental.pallas import tpu as pltpu
from jax.experimental.pallas import tpu_sc as plsc

VOCAB = 100000
EMB_DIM = 128
NUM_CLASSES = 64


_ROWS_PER_BLOCK = 4000


def _project_body(emb_ref, w_ref, b_ref, out_ref):
    out_ref[...] = (
        jnp.dot(emb_ref[...], w_ref[...], preferred_element_type=jnp.float32)
        + b_ref[...]
    )


def _project(emb_table, W, b2):
    n_blocks = VOCAB // _ROWS_PER_BLOCK
    return pl.pallas_call(
        _project_body,
        grid=(n_blocks,),
        in_specs=[
            pl.BlockSpec((_ROWS_PER_BLOCK, EMB_DIM), lambda i: (i, 0)),
            pl.BlockSpec((EMB_DIM, 2 * NUM_CLASSES), lambda i: (0, 0)),
            pl.BlockSpec((1, 2 * NUM_CLASSES), lambda i: (0, 0)),
        ],
        out_specs=pl.BlockSpec(
            (_ROWS_PER_BLOCK, 2 * NUM_CLASSES), lambda i: (i, 0)
        ),
        out_shape=jax.ShapeDtypeStruct((VOCAB, 2 * NUM_CLASSES), jnp.float32),
    )(emb_table, W, b2)


_BG = 8
_LC = 20
_TOK = _BG * _LC


def _make_gather(B, L, nw):
    n_ids = B * L
    ids_per_w = n_ids // nw
    n_chunks = ids_per_w // _TOK
    assert n_chunks % 2 == 0
    lpb = L // _LC
    row_w = L * NUM_CLASSES
    seg = _LC * NUM_CLASSES
    mesh = plsc.VectorSubcoreMesh(core_axis_name="c", subcore_axis_name="s")
    nc = mesh.num_cores

    @functools.partial(
        pl.kernel,
        mesh=mesh,
        out_type=jax.ShapeDtypeStruct((B, row_w), jnp.float32),
        scratch_types=[
            pltpu.VMEM((ids_per_w,), jnp.int32),
            pltpu.VMEM((_TOK,), jnp.int32),
            pltpu.VMEM((_TOK,), jnp.int32),
            pltpu.VMEM((_TOK, 2 * NUM_CLASSES), jnp.float32),
            pltpu.VMEM((_TOK, 2 * NUM_CLASSES), jnp.float32),
            pltpu.VMEM((_BG, seg), jnp.float32),
            pltpu.VMEM((_BG, seg), jnp.float32),
            pltpu.SemaphoreType.DMA,
            pltpu.SemaphoreType.DMA,
            pltpu.SemaphoreType.DMA,
            pltpu.SemaphoreType.DMA,
        ],
    )
    def gather_k(
        ids_hbm, p_hbm, out_hbm,
        idx_v, pbuf0, pbuf1, gbuf0, gbuf1, obuf0, obuf1,
        gsem0, gsem1, ssem0, ssem1,
    ):
        wid = lax.axis_index("s") * nc + lax.axis_index("c")
        pltpu.sync_copy(ids_hbm.at[wid], idx_v)

        def build_pbuf(j, pbuf):
            g0 = (j // lpb) * _BG * L + (j % lpb) * _LC
            for bi in range(_BG):
                src = g0 + bi * L
                dst = bi * _LC
                pbuf[pl.ds(dst, 16)] = idx_v[pl.ds(src, 16)]
                pbuf[pl.ds(dst + 4, 16)] = idx_v[pl.ds(src + 4, 16)]

        def start_gather(pbuf, gbuf, gsem):
            pltpu.async_copy(
                p_hbm.at[pbuf.at[pl.ds(0, 128)]],
                gbuf.at[pl.ds(0, 128)], gsem,
            )
            pltpu.async_copy(
                p_hbm.at[pbuf.at[pl.ds(128, _TOK - 128)]],
                gbuf.at[pl.ds(128, _TOK - 128)], gsem,
            )

        def wait_gather(pbuf, gbuf, gsem):
            pltpu.make_async_copy(
                p_hbm.at[pbuf.at[pl.ds(0, 128)]],
                gbuf.at[pl.ds(0, 128)], gsem,
            ).wait()
            pltpu.make_async_copy(
                p_hbm.at[pbuf.at[pl.ds(128, _TOK - 128)]],
                gbuf.at[pl.ds(128, _TOK - 128)], gsem,
            ).wait()

        def out_slice(j):
            g = wid * n_chunks + j
            b0 = (g // lpb) * _BG
            return out_hbm.at[
                pl.ds(b0, _BG), pl.ds((g % lpb) * seg, seg)
            ]

        def linearize(gbuf_t, obuf_t):
            for bi in range(_BG):
                def rowc(dl, c, bi=bi):
                    for k in range(NUM_CLASSES // 16):
                        obuf_t[bi, pl.ds(dl * NUM_CLASSES + 16 * k, 16)] = (
                            gbuf_t[bi * _LC + dl, pl.ds(16 * k, 16)]
                        )
                    return c

                lax.fori_loop(0, _LC, rowc, 0, unroll=5)

        build_pbuf(0, pbuf0)
        build_pbuf(1, pbuf1)
        start_gather(pbuf0, gbuf0, gsem0)
        start_gather(pbuf1, gbuf1, gsem1)

        def half_step(i, j, pbuf, gbuf, obuf, gsem, ssem):
            wait_gather(pbuf, gbuf, gsem)

            @pl.when(i > 0)
            def _():
                pltpu.make_async_copy(obuf, out_slice(j - 2), ssem).wait()

            linearize(gbuf, obuf)

            @pl.when(j + 2 < n_chunks)
            def _():
                build_pbuf(j + 2, pbuf)
                start_gather(pbuf, gbuf, gsem)

            pltpu.async_copy(obuf, out_slice(j), ssem)

        def body(i, carry):
            j = 2 * i
            half_step(i, j, pbuf0, gbuf0, obuf0, gsem0, ssem0)
            half_step(i, j + 1, pbuf1, gbuf1, obuf1, gsem1, ssem1)
            return carry

        lax.fori_loop(0, n_chunks // 2, body, 0)
        pltpu.make_async_copy(obuf0, out_slice(n_chunks - 2), ssem0).wait()
        pltpu.make_async_copy(obuf1, out_slice(n_chunks - 1), ssem1).wait()

    return gather_k


def kernel(token_ids, emb_table, W, b):
    B, L = token_ids.shape
    info = plsc.get_sparse_core_info()
    nw = info.num_cores * info.num_subcores

    W2 = jnp.concatenate([W, W], axis=1)
    b2 = jnp.concatenate([b, b]).reshape(1, 2 * NUM_CLASSES)
    proj = _project(emb_table, W2, b2)

    ids2 = token_ids.reshape(nw, (B // nw) * L).astype(jnp.int32)
    out2 = _make_gather(B, L, nw)(ids2, proj)
    return out2.T.reshape(L, NUM_CLASSES, B).transpose(2, 0, 1)
